# Initial kernel scaffold; baseline (speedup 1.0000x reference)
import jax
import jax.numpy as jnp
from jax.experimental import pallas as pl


def kernel(x, k):
    raise NotImplementedError("write your pallas kernel here")


# speedup vs baseline: 1.7828x; 1.7828x over previous
import functools

import jax
import jax.numpy as jnp
from jax import lax
from jax.experimental import pallas as pl
from jax.experimental.pallas import tpu as pltpu
from jax.experimental.pallas import tpu_sc as plsc

R, N, K = 128, 8192, 256
L = 16
NWORKERS = 32
ROWS_PER_W = R // NWORKERS
NV = N // L
KV = K // L

_i32 = jnp.int32


def _lane_iota():
    return lax.iota(_i32, L)


def _cross_stage(ks, vs, dist):
    n = len(ks)
    for bs in range(0, n, 2 * dist):
        for i in range(bs, bs + dist):
            a, b = i, i + dist
            c = ks[a] <= ks[b]
            ka2 = jnp.where(c, ks[a], ks[b])
            kb2 = jnp.where(c, ks[b], ks[a])
            va2 = jnp.where(c, vs[a], vs[b])
            vb2 = jnp.where(c, vs[b], vs[a])
            ks[a], ks[b], vs[a], vs[b] = ka2, kb2, va2, vb2


def _rev16(x):
    return lax.rev(x, (0,))


def _merge(ak, av, bk, bv):
    m = len(ak)
    ks = ak + [_rev16(bk[m - 1 - j]) for j in range(m)]
    vs = av + [_rev16(bv[m - 1 - j]) for j in range(m)]
    d = m
    while d >= 1:
        _cross_stage(ks, vs, d)
        d //= 2
    ok, ov = [], []
    for k_, v_ in zip(ks, vs):
        sk, sv = plsc.sort_key_val(k_, v_)
        ok.append(sk)
        ov.append(sv)
    return ok, ov


def _sort_kv(keys, vals):
    runs = []
    for j in range(len(keys)):
        sk, sv = plsc.sort_key_val(keys[j], vals[j])
        runs.append(([sk], [sv]))
    while len(runs) > 1:
        nxt = []
        for a in range(0, len(runs), 2):
            nxt.append(_merge(runs[a][0], runs[a][1],
                              runs[a + 1][0], runs[a + 1][1]))
        runs = nxt
    return runs[0]


def _body(x_hbm, out_hbm, rowf, keys, hist, ckey, cidx, skey, obuf):
    wid = lax.axis_index("s") * 2 + lax.axis_index("c")
    iota = _lane_iota()
    ones = jnp.full((L,), 1, _i32)

    def row_body(rr, _carry):
        row = wid * ROWS_PER_W + rr
        pltpu.sync_copy(x_hbm.at[row], rowf)

        def keybody(i, c):
            v = rowf[pl.ds(i * L, L)]
            v = jnp.where(v != v, jnp.float32(1e10), v)
            b = lax.bitcast_convert_type(v, _i32)
            m = lax.shift_right_arithmetic(b, 31) & _i32(0x7FFFFFFF)
            keys[pl.ds(i * L, L)] = b ^ m
            return c

        lax.fori_loop(0, NV, keybody, _i32(0))

        kr = _i32(K)
        c_total = _i32(0)
        pfx = _i32(0)
        for level in range(4):
            shift = 24 - 8 * level

            def zbody(i, c):
                hist[pl.ds(i * L, L)] = jnp.zeros((L,), _i32)
                return c

            lax.fori_loop(0, 4096 // L, zbody, _i32(0))

            if level == 0:
                def hbody(i, c):
                    v = keys[pl.ds(i * L, L)]
                    dig = (lax.shift_right_arithmetic(v, 24) & _i32(0xFF)) ^ _i32(0x80)
                    plsc.addupdate_scatter(hist, [iota * 256 + dig], ones)
                    return c
            else:
                hi_bits = 8 * level
                hi_mask = _i32((1 << hi_bits) - 1)
                pfx_now = pfx

                def hbody(i, c, _s=shift, _hm=hi_mask, _pf=pfx_now):
                    v = keys[pl.ds(i * L, L)]
                    hi = lax.shift_right_arithmetic(v, _s + 8) & _hm
                    dig = lax.shift_right_arithmetic(v, _s) & _i32(0xFF)
                    plsc.addupdate_scatter(hist, [iota * 256 + dig], ones,
                                           mask=hi == _pf)
                    return c

            lax.fori_loop(0, NV, hbody, _i32(0))

            def sbody(j, carry):
                nb, cb, run = carry
                tot = hist[pl.ds(j * L, L)]
                for lane in range(1, L):
                    tot = tot + hist[pl.ds(lane * 256 + j * L, L)]
                cum = plsc.cumsum(tot) + run
                lt = cum < kr
                nb = nb + jnp.sum(lt.astype(_i32))
                cb = cb + jnp.sum(jnp.where(lt, tot, _i32(0)))
                return nb, cb, jnp.max(cum)

            nb, cb, _run = lax.fori_loop(0, 16, sbody,
                                         (_i32(0), _i32(0), _i32(0)))
            c_total = c_total + cb
            kr = kr - cb
            raw = nb ^ _i32(0x80) if level == 0 else nb
            pfx = lax.shift_left(pfx, 8) | raw
        thr = pfx

        budget = _i32(K) - c_total

        def cbody(i, carry):
            base, base_eq = carry
            v = keys[pl.ds(i * L, L)]
            gidx = i * L + iota
            m_lt = v < thr
            m_eq = v == thr
            eqc = base_eq + plsc.cumsum(m_eq.astype(_i32))
            m_take = m_lt | (m_eq & (eqc <= budget))
            ti = m_take.astype(_i32)
            pos = base + plsc.cumsum(ti) - 1
            plsc.store_scatter(ckey, [pos], v, mask=m_take)
            plsc.store_scatter(cidx, [pos], gidx, mask=m_take)
            return base + jnp.sum(ti), base_eq + jnp.sum(m_eq.astype(_i32))

        lax.fori_loop(0, NV, cbody, (_i32(0), _i32(0)))

        kl = [ckey[pl.ds(j * L, L)] for j in range(KV)]
        vl = [cidx[pl.ds(j * L, L)] for j in range(KV)]
        kl, vl = _sort_kv(kl, vl)

        for j in range(KV):
            skey[pl.ds((j + 1) * L, L)] = kl[j]
        skey[pl.ds(0, L)] = kl[0]
        comp = []
        carry = _i32(0)
        for j in range(KV):
            cur = skey[pl.ds((j + 1) * L, L)]
            prv = plsc.load_gather(skey, [iota + (j * L + L - 1)])
            neq = cur != prv
            g = jnp.where(neq, j * L + iota, _i32(0))
            f = jnp.maximum(plsc.cummax(g), carry)
            carry = jnp.max(f)
            comp.append(f * N + vl[j])

        _, ol = _sort_kv(comp, vl)
        for j in range(KV):
            obuf[pl.ds(j * L, L)] = ol[j]
        pltpu.sync_copy(obuf, out_hbm.at[row])
        return _carry

    lax.fori_loop(0, ROWS_PER_W, row_body, _i32(0))


@functools.partial(jax.jit, static_argnames=())
def _run(x):
    mesh = plsc.VectorSubcoreMesh(core_axis_name="c", subcore_axis_name="s")
    fn = pl.kernel(
        _body,
        out_type=jax.ShapeDtypeStruct((R, K), _i32),
        mesh=mesh,
        compiler_params=pltpu.CompilerParams(needs_layout_passes=False),
        scratch_types=[
            pltpu.VMEM((N,), jnp.float32),
            pltpu.VMEM((N,), _i32),
            pltpu.VMEM((4096,), _i32),
            pltpu.VMEM((K,), _i32),
            pltpu.VMEM((K,), _i32),
            pltpu.VMEM((K + L,), _i32),
            pltpu.VMEM((K,), _i32),
        ],
    )
    return fn(x)


def kernel(x, k):
    del k
    return _run(x)
